# baseline (speedup 1.0000x reference)
import functools

import jax
import jax.numpy as jnp
from jax import lax
from jax.experimental import pallas as pl
from jax.experimental.pallas import tpu as pltpu
from jax.experimental.pallas import tpu_sc as plsc

ROWS = 128
COLS = 100000


_K1 = 42
_K2 = 0x1BD11BDA ^ 42
_ROT = ((13, 15, 26, 6), (17, 29, 16, 24), (13, 15, 26, 6),
        (17, 29, 16, 24), (13, 15, 26, 6))
_KA = (_K1, _K2, None, _K1, _K2)
_KB = (_K2 + 1, 0 + 2, _K1 + 3, _K2 + 4, 0 + 5)


def _threefry_xor(x1):
    x0 = x1
    for g in range(5):
        first = g == 0
        for r in _ROT[g]:
            if first:
                first = False
            else:
                x0 = x0 + x1
            x1 = (x1 << r) | (x1 >> (32 - r))
            x1 = x1 ^ x0
        if _KA[g] is not None:
            x0 = x0 + jnp.uint32(_KA[g])
        x1 = x1 + jnp.uint32(_KB[g])
    return x0 ^ x1


def _bits_to_u(bits):
    return lax.bitcast_convert_type((bits >> 9) | jnp.uint32(0x3F800000),
                                    jnp.float32) - jnp.float32(1.0)


CS = 23040
SC_CH = 5760
SC_NCH = 2
RPW = 8
SC_UNROLL = 4
SC_HALF = CS // 2


def _ln(x):
    bits = lax.bitcast_convert_type(x, jnp.uint32)
    e = (bits >> 23).astype(jnp.int32) - 127
    m = lax.bitcast_convert_type((bits & jnp.uint32(0x007FFFFF))
                                 | jnp.uint32(0x3F800000), jnp.float32)
    big = m > jnp.float32(1.4142135)
    m = jnp.where(big, m * jnp.float32(0.5), m)
    e = jnp.where(big, e + 1, e).astype(jnp.float32)
    t = m - jnp.float32(1.0)
    s = t / (t + jnp.float32(2.0))
    z = s * s
    p = jnp.float32(2.0 / 9.0)
    p = p * z + jnp.float32(2.0 / 7.0)
    p = p * z + jnp.float32(2.0 / 5.0)
    p = p * z + jnp.float32(2.0 / 3.0)
    lnm = (s * z) * p + (s + s)
    return e * jnp.float32(0.6931471805599453) + lnm


def _r16(x1_u32, lp_vec):
    return (-_ln(_bits_to_u(_threefry_xor(x1_u32)))) * jnp.exp(-lp_vec)


@functools.cache
def _make_sc():
    mesh = plsc.VectorSubcoreMesh(core_axis_name="c", subcore_axis_name="s")

    @functools.partial(
        pl.kernel, mesh=mesh,
        out_type=(jax.ShapeDtypeStruct((32, RPW, 16), jnp.int32),
                  jax.ShapeDtypeStruct((32, RPW, 16), jnp.float32)),
        scratch_types=[
            pltpu.VMEM((RPW, SC_CH), jnp.float32),
            pltpu.VMEM((RPW, SC_CH), jnp.float32),
            pltpu.VMEM((RPW, 16), jnp.float32),
            pltpu.VMEM((RPW, 16), jnp.int32),
            pltpu.SemaphoreType.DMA,
            pltpu.SemaphoreType.DMA,
        ],
    )
    def sc_kernel(lp_hbm, outi_hbm, outv_hbm, buf_a, buf_b, bvst, bist,
                  sem_a, sem_b):
        wid = lax.axis_index("s") * 2 + lax.axis_index("c")
        half = wid // 16
        grp = wid % 16
        r0 = grp * RPW
        cbase = half * SC_HALF
        lane = lax.iota(jnp.int32, 16)
        pos_inf = jnp.full((16,), jnp.inf, jnp.float32)

        for rr in range(RPW):
            bvst[rr, :] = pos_inf
            bist[rr, :] = jnp.zeros((16,), jnp.int32)

        def start(c, buf, sem):
            pltpu.async_copy(
                lp_hbm.at[pl.ds(r0, RPW), pl.ds(cbase + c * SC_CH, SC_CH)],
                buf, sem)

        def wait(buf, sem):
            pltpu.make_async_copy(
                lp_hbm.at[pl.ds(0, RPW), pl.ds(0, SC_CH)], buf, sem).wait()

        def compute(c0, buf):
            nit = SC_CH // (16 * SC_UNROLL)

            def row_body(rr, _):
                row = r0 + rr
                base_u = ((row * COLS + c0 + _K1).astype(jnp.uint32)
                          + lane.astype(jnp.uint32))
                base_i = c0 + lane

                def body(k, carry):
                    bv, bi = carry
                    for uu in range(SC_UNROLL):
                        off = k * (16 * SC_UNROLL) + uu * 16
                        lpv = buf[rr, pl.ds(off, 16)]
                        s = _r16(base_u + off.astype(jnp.uint32), lpv)
                        m = s < bv
                        bv = jnp.where(m, s, bv)
                        bi = jnp.where(m, base_i + off, bi)
                    return bv, bi

                bv, bi = lax.fori_loop(0, nit, body,
                                       (bvst[rr, :], bist[rr, :]))
                bvst[rr, :] = bv
                bist[rr, :] = bi
                return 0

            lax.fori_loop(0, RPW, row_body, 0)

        start(0, buf_a, sem_a)
        start(1, buf_b, sem_b)

        def pair(cc, _):
            c = cc * 2
            wait(buf_a, sem_a)
            compute(cbase + c * SC_CH, buf_a)

            @pl.when(c + 2 < SC_NCH)
            def _():
                start(c + 2, buf_a, sem_a)

            wait(buf_b, sem_b)
            compute(cbase + (c + 1) * SC_CH, buf_b)

            @pl.when(c + 3 < SC_NCH)
            def _():
                start(c + 3, buf_b, sem_b)

            return 0

        lax.fori_loop(0, SC_NCH // 2, pair, 0)

        for rr in range(RPW):
            bvst[rr, :] = jnp.float32(0.0) - _ln(bvst[rr, :])

        pltpu.sync_copy(bist, outi_hbm.at[wid])
        pltpu.sync_copy(bvst, outv_hbm.at[wid])

    return sc_kernel


BC = 1920
SUB = 128
TC_BLK0 = CS // BC
NBLK = (COLS - CS + BC - 1) // BC


def _tc_body(lp_ref, outv_ref, outi_ref, bv_ref, bi_ref):
    j = pl.program_id(0)

    @pl.when(j == 0)
    def _init():
        bv_ref[...] = jnp.full((ROWS, SUB), -jnp.inf, jnp.float32)
        bi_ref[...] = jnp.zeros((ROWS, SUB), jnp.int32)

    rowc = jax.lax.broadcasted_iota(jnp.int32, (ROWS, SUB), 0) * COLS
    ci = jax.lax.broadcasted_iota(jnp.int32, (ROWS, SUB), 1)
    base = rowc + ci

    for k in range(BC // SUB):
        lp = lp_ref[:, k * SUB:(k + 1) * SUB]
        c0 = CS + j * BC + k * SUB
        bits = _threefry_xor((base + (c0 + _K1)).astype(jnp.uint32))
        u = _bits_to_u(bits)
        score = lp - jnp.log(-jnp.log(u))
        score = jnp.where(ci + c0 < COLS, score, -jnp.inf)
        bv = bv_ref[...]
        m = score > bv
        bv_ref[...] = jnp.where(m, score, bv)
        bi_ref[...] = jnp.where(m, ci + c0, bi_ref[...])

    @pl.when(j == NBLK - 1)
    def _final():
        outv_ref[...] = bv_ref[...]
        outi_ref[...] = bi_ref[...]


def _tc_part(log_p):
    return pl.pallas_call(
        _tc_body,
        grid=(NBLK,),
        in_specs=[pl.BlockSpec((ROWS, BC), lambda j: (0, TC_BLK0 + j))],
        out_specs=(pl.BlockSpec((ROWS, SUB), lambda j: (0, 0)),
                   pl.BlockSpec((ROWS, SUB), lambda j: (0, 0))),
        out_shape=(jax.ShapeDtypeStruct((ROWS, SUB), jnp.float32),
                   jax.ShapeDtypeStruct((ROWS, SUB), jnp.int32)),
        scratch_shapes=[
            pltpu.VMEM((ROWS, SUB), jnp.float32),
            pltpu.VMEM((ROWS, SUB), jnp.int32),
        ],
    )(log_p)


@jax.jit
def kernel(log_p):
    sci, scv = _make_sc()(log_p[:, :CS])
    tcv, tci = _tc_part(log_p)
    scv_r = jnp.concatenate([scv[:16].reshape(ROWS, 16),
                             scv[16:].reshape(ROWS, 16)], axis=1)
    sci_r = jnp.concatenate([sci[:16].reshape(ROWS, 16),
                             sci[16:].reshape(ROWS, 16)], axis=1)
    v = jnp.concatenate([scv_r, tcv], axis=1)
    i = jnp.concatenate([sci_r, tci], axis=1)
    mx = jnp.max(v, axis=1, keepdims=True)
    cand = jnp.where(v == mx, i, jnp.int32(COLS + 1))
    return jnp.min(cand, axis=1)


# speedup vs baseline: 1.0397x; 1.0397x over previous
import functools

import jax
import jax.numpy as jnp
from jax import lax
from jax.experimental import pallas as pl
from jax.experimental.pallas import tpu as pltpu
from jax.experimental.pallas import tpu_sc as plsc

ROWS = 128
COLS = 100000


_K1 = 42
_K2 = 0x1BD11BDA ^ 42
_ROT = ((13, 15, 26, 6), (17, 29, 16, 24), (13, 15, 26, 6),
        (17, 29, 16, 24), (13, 15, 26, 6))
_KA = (_K1, _K2, None, _K1, _K2)
_KB = (_K2 + 1, 0 + 2, _K1 + 3, _K2 + 4, 0 + 5)


def _threefry_xor(x1):
    x0 = x1
    for g in range(5):
        first = g == 0
        for r in _ROT[g]:
            if first:
                first = False
            else:
                x0 = x0 + x1
            x1 = (x1 << r) | (x1 >> (32 - r))
            x1 = x1 ^ x0
        if _KA[g] is not None:
            x0 = x0 + jnp.uint32(_KA[g])
        x1 = x1 + jnp.uint32(_KB[g])
    return x0 ^ x1


def _bits_to_u(bits):
    return lax.bitcast_convert_type((bits >> 9) | jnp.uint32(0x3F800000),
                                    jnp.float32) - jnp.float32(1.0)


CS = 23040
SC_CH = 5760
SC_NCH = 2
RPW = 8
SC_UNROLL = 4
SC_HALF = CS // 2


def _ln(x):
    bits = lax.bitcast_convert_type(x, jnp.uint32)
    e = (bits >> 23).astype(jnp.int32) - 127
    m = lax.bitcast_convert_type((bits & jnp.uint32(0x007FFFFF))
                                 | jnp.uint32(0x3F800000), jnp.float32)
    big = m > jnp.float32(1.4142135)
    m = jnp.where(big, m * jnp.float32(0.5), m)
    e = jnp.where(big, e + 1, e).astype(jnp.float32)
    t = m - jnp.float32(1.0)
    s = t / (t + jnp.float32(2.0))
    z = s * s
    p = jnp.float32(2.0 / 9.0)
    p = p * z + jnp.float32(2.0 / 7.0)
    p = p * z + jnp.float32(2.0 / 5.0)
    p = p * z + jnp.float32(2.0 / 3.0)
    lnm = (s * z) * p + (s + s)
    return e * jnp.float32(0.6931471805599453) + lnm


def _r16(x1_u32, lp_vec):
    return (-_ln(_bits_to_u(_threefry_xor(x1_u32)))) * jnp.exp(-lp_vec)


@functools.cache
def _make_sc():
    mesh = plsc.VectorSubcoreMesh(core_axis_name="c", subcore_axis_name="s")

    @functools.partial(
        pl.kernel, mesh=mesh,
        out_type=(jax.ShapeDtypeStruct((32, RPW, 16), jnp.int32),
                  jax.ShapeDtypeStruct((32, RPW, 16), jnp.float32)),
        scratch_types=[
            pltpu.VMEM((RPW, SC_CH), jnp.float32),
            pltpu.VMEM((RPW, SC_CH), jnp.float32),
            pltpu.VMEM((RPW, 16), jnp.float32),
            pltpu.VMEM((RPW, 16), jnp.int32),
            pltpu.SemaphoreType.DMA,
            pltpu.SemaphoreType.DMA,
        ],
    )
    def sc_kernel(lp_hbm, outi_hbm, outv_hbm, buf_a, buf_b, bvst, bist,
                  sem_a, sem_b):
        wid = lax.axis_index("s") * 2 + lax.axis_index("c")
        half = wid // 16
        grp = wid % 16
        r0 = grp * RPW
        cbase = half * SC_HALF
        lane = lax.iota(jnp.int32, 16)
        pos_inf = jnp.full((16,), jnp.inf, jnp.float32)

        for rr in range(RPW):
            bvst[rr, :] = pos_inf
            bist[rr, :] = jnp.zeros((16,), jnp.int32)

        def start(c, buf, sem):
            pltpu.async_copy(
                lp_hbm.at[pl.ds(r0, RPW), pl.ds(cbase + c * SC_CH, SC_CH)],
                buf, sem)

        def wait(buf, sem):
            pltpu.make_async_copy(
                lp_hbm.at[pl.ds(0, RPW), pl.ds(0, SC_CH)], buf, sem).wait()

        def compute(c0, buf):
            nit = SC_CH // (16 * SC_UNROLL)

            def row_body(rr, _):
                row = r0 + rr
                base_u = ((row * COLS + c0 + _K1).astype(jnp.uint32)
                          + lane.astype(jnp.uint32))
                base_i = c0 + lane

                def body(k, carry):
                    bv, bi = carry
                    for uu in range(SC_UNROLL):
                        off = k * (16 * SC_UNROLL) + uu * 16
                        lpv = buf[rr, pl.ds(off, 16)]
                        s = _r16(base_u + off.astype(jnp.uint32), lpv)
                        m = s < bv
                        bv = jnp.where(m, s, bv)
                        bi = jnp.where(m, base_i + off, bi)
                    return bv, bi

                bv, bi = lax.fori_loop(0, nit, body,
                                       (bvst[rr, :], bist[rr, :]))
                bvst[rr, :] = bv
                bist[rr, :] = bi
                return 0

            lax.fori_loop(0, RPW, row_body, 0)

        start(0, buf_a, sem_a)
        start(1, buf_b, sem_b)

        def pair(cc, _):
            c = cc * 2
            wait(buf_a, sem_a)
            compute(cbase + c * SC_CH, buf_a)

            @pl.when(c + 2 < SC_NCH)
            def _():
                start(c + 2, buf_a, sem_a)

            wait(buf_b, sem_b)
            compute(cbase + (c + 1) * SC_CH, buf_b)

            @pl.when(c + 3 < SC_NCH)
            def _():
                start(c + 3, buf_b, sem_b)

            return 0

        lax.fori_loop(0, SC_NCH // 2, pair, 0)

        for rr in range(RPW):
            bvst[rr, :] = jnp.float32(0.0) - _ln(bvst[rr, :])

        pltpu.sync_copy(bist, outi_hbm.at[wid])
        pltpu.sync_copy(bvst, outv_hbm.at[wid])

    return sc_kernel


BC = 2048
SUB = 128
TC_FULL = (COLS - CS) // BC
TC_TAIL = COLS - CS - TC_FULL * BC


def _tc_fold(buf, c0, ncols, base, rowc_ci, bv_ref, bi_ref):
    ci = rowc_ci
    for k in range(ncols // SUB):
        _tc_fold1(buf[:, k * SUB:(k + 1) * SUB], c0 + k * SUB, base, ci,
                  bv_ref, bi_ref, SUB)
    rem = ncols % SUB
    if rem:
        _tc_fold1(buf[:, ncols - rem:ncols], c0 + ncols - rem, base[:, :rem],
                  ci[:, :rem], bv_ref, bi_ref, rem)


def _tc_fold1(lp, c0, base, ci, bv_ref, bi_ref, width):
    bits = _threefry_xor((base + (c0 + _K1)).astype(jnp.uint32))
    u = _bits_to_u(bits)
    score = lp - jnp.log(-jnp.log(u))
    if width == SUB:
        bv = bv_ref[...]
        m = score > bv
        bv_ref[...] = jnp.where(m, score, bv)
        bi_ref[...] = jnp.where(m, ci + c0, bi_ref[...])
    else:
        bv = bv_ref[:, :width]
        m = score > bv
        bv_ref[:, :width] = jnp.where(m, score, bv)
        bi_ref[:, :width] = jnp.where(m, ci + c0, bi_ref[:, :width])


def _tc_body(lp_hbm, outv_ref, outi_ref, buf_a, buf_b, buf_t, bv_ref,
             bi_ref, sem_a, sem_b, sem_t):
    bv_ref[...] = jnp.full((ROWS, SUB), -jnp.inf, jnp.float32)
    bi_ref[...] = jnp.zeros((ROWS, SUB), jnp.int32)
    rowc = jax.lax.broadcasted_iota(jnp.int32, (ROWS, SUB), 0) * COLS
    ci = jax.lax.broadcasted_iota(jnp.int32, (ROWS, SUB), 1)
    base = rowc + ci

    def start(c, buf, sem):
        pltpu.make_async_copy(
            lp_hbm.at[:, pl.ds(CS + c * BC, BC)], buf, sem).start()

    def wait(buf, sem):
        pltpu.make_async_copy(
            lp_hbm.at[:, pl.ds(CS, BC)], buf, sem).wait()

    start(0, buf_a, sem_a)
    start(1, buf_b, sem_b)
    pltpu.make_async_copy(lp_hbm.at[:, pl.ds(COLS - TC_TAIL, TC_TAIL)],
                          buf_t, sem_t).start()

    def pair(cc, _):
        c = cc * 2
        wait(buf_a, sem_a)
        _tc_fold(buf_a, CS + c * BC, BC, base, ci, bv_ref, bi_ref)

        @pl.when(c + 2 < TC_FULL)
        def _():
            start(c + 2, buf_a, sem_a)

        wait(buf_b, sem_b)
        _tc_fold(buf_b, CS + (c + 1) * BC, BC, base, ci, bv_ref, bi_ref)

        @pl.when(c + 3 < TC_FULL)
        def _():
            start(c + 3, buf_b, sem_b)

        return 0

    jax.lax.fori_loop(0, TC_FULL // 2, pair, 0)
    wait(buf_a, sem_a)
    _tc_fold(buf_a, CS + (TC_FULL - 1) * BC, BC, base, ci, bv_ref, bi_ref)
    pltpu.make_async_copy(lp_hbm.at[:, pl.ds(COLS - TC_TAIL, TC_TAIL)],
                          buf_t, sem_t).wait()
    _tc_fold(buf_t, COLS - TC_TAIL, TC_TAIL, base, ci, bv_ref, bi_ref)

    outv_ref[...] = bv_ref[...]
    outi_ref[...] = bi_ref[...]


def _tc_part(log_p):
    return pl.pallas_call(
        _tc_body,
        in_specs=[pl.BlockSpec(memory_space=pl.ANY)],
        out_shape=(jax.ShapeDtypeStruct((ROWS, SUB), jnp.float32),
                   jax.ShapeDtypeStruct((ROWS, SUB), jnp.int32)),
        scratch_shapes=[
            pltpu.VMEM((ROWS, BC), jnp.float32),
            pltpu.VMEM((ROWS, BC), jnp.float32),
            pltpu.VMEM((ROWS, TC_TAIL), jnp.float32),
            pltpu.VMEM((ROWS, SUB), jnp.float32),
            pltpu.VMEM((ROWS, SUB), jnp.int32),
            pltpu.SemaphoreType.DMA,
            pltpu.SemaphoreType.DMA,
            pltpu.SemaphoreType.DMA,
        ],
    )(log_p)


@jax.jit
def kernel(log_p):
    sci, scv = _make_sc()(log_p)
    tcv, tci = _tc_part(log_p)
    scv_r = jnp.concatenate([scv[:16].reshape(ROWS, 16),
                             scv[16:].reshape(ROWS, 16)], axis=1)
    sci_r = jnp.concatenate([sci[:16].reshape(ROWS, 16),
                             sci[16:].reshape(ROWS, 16)], axis=1)
    v = jnp.concatenate([scv_r, tcv], axis=1)
    i = jnp.concatenate([sci_r, tci], axis=1)
    mx = jnp.max(v, axis=1, keepdims=True)
    cand = jnp.where(v == mx, i, jnp.int32(COLS + 1))
    return jnp.min(cand, axis=1)
